# baseline (speedup 1.0000x reference)
import jax
import jax.numpy as jnp
from jax import lax
from jax.experimental import pallas as pl
from jax.experimental.pallas import tpu as pltpu
from jax.experimental.pallas import tpu_sc as plsc
import functools

LANES = 128
NC = 2
NS = 16
NW = NC * NS


def _sc_aggregate(x, src3, dst3, zrows, zdeg, n_pad, n_chunks):
    d = x.shape[1]
    rows_per_tile = n_pad // NS
    mesh = plsc.VectorSubcoreMesh(core_axis_name="c", subcore_axis_name="s")

    @functools.partial(
        pl.kernel,
        out_type=(
            jax.ShapeDtypeStruct((NC, n_pad, d), jnp.float32),
            jax.ShapeDtypeStruct((n_pad,), jnp.float32),
            jax.ShapeDtypeStruct((n_pad,), jnp.float32),
        ),
        mesh=mesh,
    scratch_types=[
            pltpu.VMEM((2, LANES), jnp.int32),
            pltpu.VMEM((n_chunks, LANES), jnp.int32),
            pltpu.VMEM((LANES, d), jnp.float32),
            pltpu.VMEM((LANES, d), jnp.float32),
            pltpu.VMEM((LANES,), jnp.float32),
            pltpu.SemaphoreType.DMA,
            pltpu.SemaphoreType.DMA,
            pltpu.SemaphoreType.DMA,
            pltpu.SemaphoreType.DMA,
            pltpu.SemaphoreType.DMA,
            pltpu.SemaphoreType.DMA,
            pltpu.VMEM_SHARED((n_pad, d), jnp.float32),
            pltpu.VMEM_SHARED((n_pad,), jnp.float32),
        ],
    )
    def agg_kernel(x_hbm, src_hbm, dst_hbm, zrows_hbm, zdeg_hbm,
                   aggp_hbm, deg0_hbm, deg1_hbm,
                   src_c, dst_v, rows_a, rows_b, ones_v, sem_a, sem_b,
                   isem, rsem_a, rsem_b, dsem, agg_sh, deg_sh):
        cid = lax.axis_index("c")
        sid = lax.axis_index("s")
        wid = cid * NS + sid
        base = sid * rows_per_tile

        pltpu.sync_copy(zrows_hbm, agg_sh.at[pl.ds(base, rows_per_tile)])
        pltpu.sync_copy(zdeg_hbm, deg_sh.at[pl.ds(base, rows_per_tile)])

        pltpu.sync_copy(dst_hbm.at[wid], dst_v)

        for j in range(LANES // 16):
            ones_v[pl.ds(j * 16, 16)] = jnp.ones((16,), jnp.float32)

        plsc.subcore_barrier()

        bufs = (rows_a, rows_b)
        gsems = (sem_a, sem_b)
        rsems = (rsem_a, rsem_b)
        pltpu.sync_copy(src_hbm.at[wid, 0], src_c.at[0])
        if n_chunks > 1:
            pltpu.async_copy(src_hbm.at[wid, 1], src_c.at[1], isem)

        quarter = LANES // 8

        def sub_gather(o, slot, buf, gsem):
            pltpu.async_copy(x_hbm.at[src_c.at[slot, pl.ds(o, quarter)]],
                             buf.at[pl.ds(o, quarter)], gsem)

        def sub_wait(o, slot, buf, gsem):
            pltpu.make_async_copy(
                x_hbm.at[src_c.at[slot, pl.ds(o, quarter)]],
                buf.at[pl.ds(o, quarter)], gsem).wait()

        for o in range(0, LANES, quarter):
            sub_gather(o, 0, rows_a, sem_a)

        def step(j, p, first=False):
            buf, nbuf = bufs[p], bufs[1 - p]

            @pl.when(j + 1 < n_chunks)
            def _():
                if not first:
                    pltpu.make_async_copy(
                        nbuf, agg_sh.at[dst_v.at[0]], rsems[1 - p]).wait()
                pltpu.make_async_copy(
                    src_hbm.at[wid, 0], src_c.at[1 - p], isem).wait()

            for o in range(0, LANES, LANES // 8):
                sub_wait(o, p, buf, gsems[p])

                @pl.when(j + 1 < n_chunks)
                def _():
                    sub_gather(o, 1 - p, nbuf, gsems[1 - p])

            @pl.when(j + 2 < n_chunks)
            def _():
                pltpu.async_copy(src_hbm.at[wid, j + 2], src_c.at[p], isem)

            pltpu.async_copy(buf, agg_sh.at[dst_v.at[j]], rsems[p],
                             add=True)
            pltpu.async_copy(ones_v, deg_sh.at[dst_v.at[j]], dsem,
                             add=True)

        step(0, 0, first=True)

        def pair(jj, carry):
            step(jj * 2 + 1, 1)
            step(jj * 2 + 2, 0)
            return carry

        lax.fori_loop(0, (n_chunks - 1) // 2, pair, 0)
        for j in range(n_chunks - 1 - 2 * ((n_chunks - 1) // 2), 0, -1):
            step(n_chunks - j, (n_chunks - j) % 2)

        for p in (0, 1):
            if n_chunks > p:
                pltpu.make_async_copy(
                    bufs[p], agg_sh.at[dst_v.at[0]], rsems[p]).wait()

        def drain(j, carry):
            pltpu.make_async_copy(ones_v, deg_sh.at[dst_v.at[0]],
                                  dsem).wait()
            return carry

        lax.fori_loop(0, n_chunks, drain, 0)

        plsc.subcore_barrier()

        pltpu.sync_copy(agg_sh.at[pl.ds(base, rows_per_tile)],
                        aggp_hbm.at[cid, pl.ds(base, rows_per_tile)])

        @pl.when(cid == 0)
        def _():
            pltpu.sync_copy(deg_sh.at[pl.ds(base, rows_per_tile)],
                            deg0_hbm.at[pl.ds(base, rows_per_tile)])

        @pl.when(cid == 1)
        def _():
            pltpu.sync_copy(deg_sh.at[pl.ds(base, rows_per_tile)],
                            deg1_hbm.at[pl.ds(base, rows_per_tile)])

    return agg_kernel(x, src3, dst3, zrows, zdeg)


def _tc_finalize(agg_part, deg0c, deg1c, x, w1t, w2t, b2):
    n, d = x.shape
    blk = 1000
    grid = n // blk

    def body(a0, a1, d0, d1, xr, w1, w2, br, o):
        deg = jnp.maximum(d0[...] + d1[...], 1.0)
        agg = (a0[0] + a1[0]) / deg
        acc = jnp.dot(agg, w1[...], preferred_element_type=jnp.float32)
        acc = acc + jnp.dot(xr[...], w2[...],
                            preferred_element_type=jnp.float32)
        o[...] = acc + br[...]

    return pl.pallas_call(
        body,
        grid=(grid,),
        in_specs=[
            pl.BlockSpec((1, blk, d), lambda i: (0, i, 0)),
            pl.BlockSpec((1, blk, d), lambda i: (1, i, 0)),
            pl.BlockSpec((blk, 1), lambda i: (i, 0)),
            pl.BlockSpec((blk, 1), lambda i: (i, 0)),
            pl.BlockSpec((blk, d), lambda i: (i, 0)),
            pl.BlockSpec((d, d), lambda i: (0, 0)),
            pl.BlockSpec((d, d), lambda i: (0, 0)),
            pl.BlockSpec((1, d), lambda i: (0, 0)),
        ],
        out_specs=pl.BlockSpec((blk, d), lambda i: (i, 0)),
        out_shape=jax.ShapeDtypeStruct((n, d), jnp.float32),
    )(agg_part, agg_part, deg0c, deg1c, x, w1t, w2t, b2)


def kernel(x, edge_index, num_nodes, W, b):
    n, d = x.shape
    e = edge_index.shape[1]

    n_chunks = -(-e // (NW * LANES))
    e_pad = NW * n_chunks * LANES
    pad = e_pad - e
    src = edge_index[0]
    dst = edge_index[1]
    if pad:
        src = jnp.concatenate([src, jnp.zeros((pad,), jnp.int32)])
        dst = jnp.concatenate([dst, jnp.full((pad,), n, jnp.int32)])
    src3 = src.reshape(NW, n_chunks, LANES)
    dst3 = dst.reshape(NW, n_chunks, LANES)

    rows_per_tile = -(-(n + 1) // (NS * LANES)) * LANES
    n_pad = rows_per_tile * NS

    zrows = jnp.zeros((rows_per_tile, d), jnp.float32)
    zdeg = jnp.zeros((rows_per_tile,), jnp.float32)

    agg_part, deg0, deg1 = _sc_aggregate(x, src3, dst3, zrows, zdeg,
                                         n_pad, n_chunks)

    w1t = W[:, :d].T
    w2t = W[:, d:].T
    b2 = b.reshape(1, d)
    return _tc_finalize(agg_part, deg0.reshape(n_pad, 1),
                        deg1.reshape(n_pad, 1), x, w1t, w2t, b2)


# speedup vs baseline: 1.0014x; 1.0014x over previous
import jax
import jax.numpy as jnp
from jax import lax
from jax.experimental import pallas as pl
from jax.experimental.pallas import tpu as pltpu
from jax.experimental.pallas import tpu_sc as plsc
import functools

LANES = 128
NC = 2
NS = 16
NW = NC * NS


def _sc_aggregate(x, src3, dst3, zrows, zdeg, n_pad, n_chunks):
    d = x.shape[1]
    rows_per_tile = n_pad // NS
    mesh = plsc.VectorSubcoreMesh(core_axis_name="c", subcore_axis_name="s")

    @functools.partial(
        pl.kernel,
        out_type=(
            jax.ShapeDtypeStruct((NC, n_pad, d), jnp.float32),
            jax.ShapeDtypeStruct((n_pad,), jnp.float32),
            jax.ShapeDtypeStruct((n_pad,), jnp.float32),
        ),
        mesh=mesh,
    scratch_types=[
            pltpu.VMEM((2, LANES), jnp.int32),
            pltpu.VMEM((n_chunks, LANES), jnp.int32),
            pltpu.VMEM((LANES, d), jnp.float32),
            pltpu.VMEM((LANES, d), jnp.float32),
            pltpu.VMEM((LANES,), jnp.float32),
            pltpu.SemaphoreType.DMA,
            pltpu.SemaphoreType.DMA,
            pltpu.SemaphoreType.DMA,
            pltpu.SemaphoreType.DMA,
            pltpu.SemaphoreType.DMA,
            pltpu.SemaphoreType.DMA,
            pltpu.VMEM_SHARED((n_pad, d), jnp.float32),
            pltpu.VMEM_SHARED((n_pad,), jnp.float32),
        ],
    )
    def agg_kernel(x_hbm, src_hbm, dst_hbm, zrows_hbm, zdeg_hbm,
                   aggp_hbm, deg0_hbm, deg1_hbm,
                   src_c, dst_v, rows_a, rows_b, ones_v, sem_a, sem_b,
                   isem, rsem_a, rsem_b, dsem, agg_sh, deg_sh):
        cid = lax.axis_index("c")
        sid = lax.axis_index("s")
        wid = cid * NS + sid
        base = sid * rows_per_tile

        pltpu.sync_copy(zrows_hbm, agg_sh.at[pl.ds(base, rows_per_tile)])
        pltpu.sync_copy(zdeg_hbm, deg_sh.at[pl.ds(base, rows_per_tile)])

        pltpu.sync_copy(dst_hbm.at[wid], dst_v)

        for j in range(LANES // 16):
            ones_v[pl.ds(j * 16, 16)] = jnp.ones((16,), jnp.float32)

        plsc.subcore_barrier()

        bufs = (rows_a, rows_b)
        gsems = (sem_a, sem_b)
        rsems = (rsem_a, rsem_b)
        pltpu.sync_copy(src_hbm.at[wid, 0], src_c.at[0])
        if n_chunks > 1:
            pltpu.async_copy(src_hbm.at[wid, 1], src_c.at[1], isem)

        quarter = LANES // 4

        def sub_gather(o, slot, buf, gsem):
            pltpu.async_copy(x_hbm.at[src_c.at[slot, pl.ds(o, quarter)]],
                             buf.at[pl.ds(o, quarter)], gsem)

        def sub_wait(o, slot, buf, gsem):
            pltpu.make_async_copy(
                x_hbm.at[src_c.at[slot, pl.ds(o, quarter)]],
                buf.at[pl.ds(o, quarter)], gsem).wait()

        for o in range(0, LANES, quarter):
            sub_gather(o, 0, rows_a, sem_a)

        def step(j, p, first=False):
            buf, nbuf = bufs[p], bufs[1 - p]

            @pl.when(j + 1 < n_chunks)
            def _():
                if not first:
                    pltpu.make_async_copy(
                        nbuf, agg_sh.at[dst_v.at[0]], rsems[1 - p]).wait()
                pltpu.make_async_copy(
                    src_hbm.at[wid, 0], src_c.at[1 - p], isem).wait()

            for o in range(0, LANES, LANES // 4):
                sub_wait(o, p, buf, gsems[p])

                @pl.when(j + 1 < n_chunks)
                def _():
                    sub_gather(o, 1 - p, nbuf, gsems[1 - p])

            @pl.when(j + 2 < n_chunks)
            def _():
                pltpu.async_copy(src_hbm.at[wid, j + 2], src_c.at[p], isem)

            pltpu.async_copy(buf, agg_sh.at[dst_v.at[j]], rsems[p],
                             add=True)
            pltpu.async_copy(ones_v, deg_sh.at[dst_v.at[j]], dsem,
                             add=True)

        step(0, 0, first=True)

        def pair(jj, carry):
            step(jj * 2 + 1, 1)
            step(jj * 2 + 2, 0)
            return carry

        lax.fori_loop(0, (n_chunks - 1) // 2, pair, 0)
        for j in range(n_chunks - 1 - 2 * ((n_chunks - 1) // 2), 0, -1):
            step(n_chunks - j, (n_chunks - j) % 2)

        for p in (0, 1):
            if n_chunks > p:
                pltpu.make_async_copy(
                    bufs[p], agg_sh.at[dst_v.at[0]], rsems[p]).wait()

        def drain(j, carry):
            pltpu.make_async_copy(ones_v, deg_sh.at[dst_v.at[0]],
                                  dsem).wait()
            return carry

        lax.fori_loop(0, n_chunks, drain, 0)

        plsc.subcore_barrier()

        pltpu.sync_copy(agg_sh.at[pl.ds(base, rows_per_tile)],
                        aggp_hbm.at[cid, pl.ds(base, rows_per_tile)])

        @pl.when(cid == 0)
        def _():
            pltpu.sync_copy(deg_sh.at[pl.ds(base, rows_per_tile)],
                            deg0_hbm.at[pl.ds(base, rows_per_tile)])

        @pl.when(cid == 1)
        def _():
            pltpu.sync_copy(deg_sh.at[pl.ds(base, rows_per_tile)],
                            deg1_hbm.at[pl.ds(base, rows_per_tile)])

    return agg_kernel(x, src3, dst3, zrows, zdeg)


def _tc_finalize(agg_part, deg0c, deg1c, x, w1t, w2t, b2):
    n, d = x.shape
    blk = 1000
    grid = n // blk

    def body(a0, a1, d0, d1, xr, w1, w2, br, o):
        deg = jnp.maximum(d0[...] + d1[...], 1.0)
        agg = (a0[0] + a1[0]) / deg
        acc = jnp.dot(agg, w1[...], preferred_element_type=jnp.float32)
        acc = acc + jnp.dot(xr[...], w2[...],
                            preferred_element_type=jnp.float32)
        o[...] = acc + br[...]

    return pl.pallas_call(
        body,
        grid=(grid,),
        in_specs=[
            pl.BlockSpec((1, blk, d), lambda i: (0, i, 0)),
            pl.BlockSpec((1, blk, d), lambda i: (1, i, 0)),
            pl.BlockSpec((blk, 1), lambda i: (i, 0)),
            pl.BlockSpec((blk, 1), lambda i: (i, 0)),
            pl.BlockSpec((blk, d), lambda i: (i, 0)),
            pl.BlockSpec((d, d), lambda i: (0, 0)),
            pl.BlockSpec((d, d), lambda i: (0, 0)),
            pl.BlockSpec((1, d), lambda i: (0, 0)),
        ],
        out_specs=pl.BlockSpec((blk, d), lambda i: (i, 0)),
        out_shape=jax.ShapeDtypeStruct((n, d), jnp.float32),
    )(agg_part, agg_part, deg0c, deg1c, x, w1t, w2t, b2)


def kernel(x, edge_index, num_nodes, W, b):
    n, d = x.shape
    e = edge_index.shape[1]

    n_chunks = -(-e // (NW * LANES))
    e_pad = NW * n_chunks * LANES
    pad = e_pad - e
    src = edge_index[0]
    dst = edge_index[1]
    if pad:
        src = jnp.concatenate([src, jnp.zeros((pad,), jnp.int32)])
        dst = jnp.concatenate([dst, jnp.full((pad,), n, jnp.int32)])
    src3 = src.reshape(NW, n_chunks, LANES)
    dst3 = dst.reshape(NW, n_chunks, LANES)

    rows_per_tile = -(-(n + 1) // (NS * LANES)) * LANES
    n_pad = rows_per_tile * NS

    zrows = jnp.zeros((rows_per_tile, d), jnp.float32)
    zdeg = jnp.zeros((rows_per_tile,), jnp.float32)

    agg_part, deg0, deg1 = _sc_aggregate(x, src3, dst3, zrows, zdeg,
                                         n_pad, n_chunks)

    w1t = W[:, :d].T
    w2t = W[:, d:].T
    b2 = b.reshape(1, d)
    return _tc_finalize(agg_part, deg0.reshape(n_pad, 1),
                        deg1.reshape(n_pad, 1), x, w1t, w2t, b2)
